# baseline (speedup 1.0000x reference)
import functools

import jax
import jax.numpy as jnp
from jax import lax
from jax.experimental import pallas as pl
from jax.experimental.pallas import tpu as pltpu
from jax.experimental.pallas import tpu_sc as plsc

NUM_HIDDENS = 128
MAX_LEN = 32768
N_IDS = 16384

_NC = 2
_NS = 16
_NW = _NC * _NS
_B_PER_W = N_IDS // _NW
_CHUNK = 128
_NCHUNKS = _B_PER_W // _CHUNK

_mesh = plsc.VectorSubcoreMesh(core_axis_name="c", subcore_axis_name="s")


@functools.partial(
    pl.kernel,
    mesh=_mesh,
    out_type=jax.ShapeDtypeStruct((N_IDS, NUM_HIDDENS), jnp.float32),
    scratch_types=[
        pltpu.VMEM((_B_PER_W,), jnp.int32),
        pltpu.VMEM((_B_PER_W, NUM_HIDDENS), jnp.float32),
        pltpu.SemaphoreType.DMA((_NCHUNKS,)),
        pltpu.SemaphoreType.DMA,
    ],
)
def _gather_rows(table_hbm, idx_hbm, out_hbm, idx_v, rows_v, gsem, wsem):
    wid = lax.axis_index("s") * _NC + lax.axis_index("c")
    base = wid * _B_PER_W
    pltpu.sync_copy(idx_hbm.at[pl.ds(base, _B_PER_W)], idx_v)
    gathers = [
        pltpu.async_copy(
            table_hbm.at[idx_v.at[pl.ds(j * _CHUNK, _CHUNK)]],
            rows_v.at[pl.ds(j * _CHUNK, _CHUNK)],
            gsem.at[j],
        )
        for j in range(_NCHUNKS)
    ]
    writes = []
    for j in range(_NCHUNKS):
        gathers[j].wait()
        writes.append(
            pltpu.async_copy(
                rows_v.at[pl.ds(j * _CHUNK, _CHUNK)],
                out_hbm.at[pl.ds(base + j * _CHUNK, _CHUNK)],
                wsem,
            )
        )
    for w in writes:
        w.wait()


def kernel(position_ids, P):
    table = P.reshape(MAX_LEN, NUM_HIDDENS)
    out = _gather_rows(table, position_ids)
    return out.reshape(1, N_IDS, NUM_HIDDENS)


# speedup vs baseline: 1.0427x; 1.0427x over previous
import jax
import jax.numpy as jnp
from jax import lax
from jax.experimental import pallas as pl

NUM_HIDDENS = 128
MAX_LEN = 32768
N_IDS = 16384

_GRID = 16
_BLK = N_IDS // _GRID

_P1 = 201.0 * 2.0**-7
_P2 = 253.0 * 2.0**-19
_P3 = 1.2675907965393353e-06
_TWO_OVER_PI = 0.6366197723675814
_RND = 12582912.0

_S1, _S2, _S3, _S4 = -1.6666667163e-01, 8.3333337680e-03, -1.9841270114e-04, 2.7557314297e-06
_C1, _C2, _C3, _C4 = -0.5, 4.1666667908e-02, -1.3888889225e-03, 2.4801587642e-05


def _tc_body(pos_ref, inv_ref, par_ref, out_ref):
    posf = pos_ref[...].astype(jnp.float32)
    y = posf * inv_ref[...]
    kf = (y * _TWO_OVER_PI + _RND) - _RND
    ki = kf.astype(jnp.int32)
    r = ((y - kf * _P1) - kf * _P2) - kf * _P3
    r2 = r * r
    sinp = r * (1.0 + r2 * (_S1 + r2 * (_S2 + r2 * (_S3 + r2 * _S4))))
    cosp = 1.0 + r2 * (_C1 + r2 * (_C2 + r2 * (_C3 + r2 * _C4)))
    q = (ki + par_ref[...]) & 3
    t = jnp.where((q & 1) == 1, cosp, sinp)
    out_ref[...] = jnp.where((q & 2) == 2, -t, t)


def kernel(position_ids, P):
    del P
    div = jnp.power(
        10000.0,
        jnp.arange(0, NUM_HIDDENS, 2, dtype=jnp.float32) / NUM_HIDDENS,
    )
    inv_full = jnp.repeat(1.0 / div, 2)[None, :]
    parity = (jnp.arange(NUM_HIDDENS, dtype=jnp.int32) & 1)[None, :]
    pos2d = position_ids[:, None]
    out = pl.pallas_call(
        _tc_body,
        grid=(_GRID,),
        in_specs=[
            pl.BlockSpec((_BLK, 1), lambda i: (i, 0)),
            pl.BlockSpec((1, NUM_HIDDENS), lambda i: (0, 0)),
            pl.BlockSpec((1, NUM_HIDDENS), lambda i: (0, 0)),
        ],
        out_specs=pl.BlockSpec((_BLK, NUM_HIDDENS), lambda i: (i, 0)),
        out_shape=jax.ShapeDtypeStruct((N_IDS, NUM_HIDDENS), jnp.float32),
    )(pos2d, inv_full, parity)
    return out.reshape(1, N_IDS, NUM_HIDDENS)
